# baseline (speedup 1.0000x reference)
import functools

import jax
import jax.numpy as jnp
from jax import lax
from jax.experimental import pallas as pl
from jax.experimental.pallas import tpu as pltpu
from jax.experimental.pallas import tpu_sc as plsc

M = 1000
A = 7
D = 256
K = 3
L = 16
MP = 1024
BLK = 128


def _dense_body(q_ref, bank_ref, sims_ref):
    g = pl.program_id(0)
    x = bank_ref[...]
    s = jnp.sum(x, axis=1)
    qv = q_ref[0]
    dot = jnp.sum(s * qv[None, :], axis=1)
    sq = jnp.sum(s * s, axis=1)
    qn = jnp.maximum(jnp.sqrt(jnp.sum(qv * qv)), 1e-8)
    norm = jnp.maximum(jnp.sqrt(sq) * (1.0 / A), 1e-8)
    sims = (dot * (1.0 / A)) / (norm * qn)
    flat = g * BLK + lax.broadcasted_iota(jnp.int32, (BLK,), 0)
    sims = jnp.where(flat < M, sims, jnp.float32(-jnp.inf))
    sims_ref[...] = sims.reshape(1, 1, BLK)


_dense_call = pl.pallas_call(
    _dense_body,
    grid=(MP // BLK,),
    in_specs=[
        pl.BlockSpec((1, D), lambda g: (0, 0)),
        pl.BlockSpec((BLK, A, D), lambda g: (g, 0, 0)),
    ],
    out_specs=pl.BlockSpec((1, 1, BLK), lambda g: (g, 0, 0)),
    out_shape=jax.ShapeDtypeStruct((MP // BLK, 1, BLK), jnp.float32),
)


_MESH = plsc.VectorSubcoreMesh(core_axis_name="c", subcore_axis_name="s")


@functools.partial(
    pl.kernel,
    out_type=(
        jax.ShapeDtypeStruct((L,), jnp.float32),
        jax.ShapeDtypeStruct((L,), jnp.int32),
    ),
    mesh=_MESH,
    scratch_types=[
        pltpu.VMEM((MP,), jnp.float32),
        pltpu.VMEM((L,), jnp.float32),
        pltpu.VMEM((L,), jnp.int32),
    ],
    compiler_params=pltpu.CompilerParams(needs_layout_passes=False),
)
def _top_kernel(sims_hbm, tv_hbm, ti_hbm, sims_v, tv_v, ti_v):
    wid = lax.axis_index("s") * 2 + lax.axis_index("c")

    @pl.when(wid == 0)
    def _():
        pltpu.sync_copy(sims_hbm, sims_v)

        lanes = lax.iota(jnp.int32, L)
        neg = jnp.float32(-jnp.inf)
        big = jnp.int32(2**30)
        mx0 = jnp.full((L,), neg)
        mi0 = jnp.full((L,), big)

        def track(v, gidx, mx, mi):
            better = v > mx
            return jnp.maximum(mx, v), jnp.where(better, gidx, mi)

        def finish(mx, mi):
            gmax = jnp.max(mx)
            gidx = jnp.min(jnp.where(mx == gmax, mi, big))
            return gmax, gidx

        def first_body(i, carry):
            v = sims_v[pl.ds(i * L, L)]
            return track(v, lanes + i * L, *carry)

        p0 = finish(*lax.fori_loop(0, MP // L, first_body, (mx0, mi0)))

        def select_next(prev):
            gv, gi = prev

            def pass_body(i, carry):
                v = sims_v[pl.ds(i * L, L)]
                gidx = lanes + i * L
                keep = (v < gv) | ((v == gv) & (gidx > gi))
                v = jnp.where(keep, v, neg)
                return track(v, gidx, *carry)

            return finish(*lax.fori_loop(0, MP // L, pass_body, (mx0, mi0)))

        p1 = select_next(p0)
        p2 = select_next(p1)

        tv = jnp.where(lanes == 0, p0[0],
                       jnp.where(lanes == 1, p1[0],
                                 jnp.where(lanes == 2, p2[0],
                                           jnp.float32(0.0))))
        ti = jnp.where(lanes == 0, p0[1],
                       jnp.where(lanes == 1, p1[1],
                                 jnp.where(lanes == 2, p2[1],
                                           jnp.int32(0))))
        tv_v[...] = tv
        ti_v[...] = ti
        pltpu.sync_copy(tv_v, tv_hbm)
        pltpu.sync_copy(ti_v, ti_hbm)


def _gather_body(ti_ref, bank_ref, best_ref):
    best_ref[...] = bank_ref[...]


_gather_call = pl.pallas_call(
    _gather_body,
    grid_spec=pltpu.PrefetchScalarGridSpec(
        num_scalar_prefetch=1,
        grid=(1,),
        in_specs=[
            pl.BlockSpec((1, A, D), lambda i, ti_ref: (ti_ref[0], 0, 0)),
        ],
        out_specs=pl.BlockSpec((1, A, D), lambda i, ti_ref: (0, 0, 0)),
    ),
    out_shape=jax.ShapeDtypeStruct((1, A, D), jnp.float32),
)


def kernel(query_embedding, memory_bank, k):
    sims8 = _dense_call(query_embedding.reshape(1, D), memory_bank)
    sims1 = sims8.reshape(MP)
    tv, ti = _top_kernel(sims1)
    best = _gather_call(ti, memory_bank)
    return (sims1[:M], tv[:K], ti[:K], best[0])


# speedup vs baseline: 1.0886x; 1.0886x over previous
import functools

import jax
import jax.numpy as jnp
from jax import lax
from jax.experimental import pallas as pl
from jax.experimental.pallas import tpu as pltpu
from jax.experimental.pallas import tpu_sc as plsc

M = 1000
A = 7
D = 256
K = 3
L = 16
MP = 1024
BLK = 128


NBLK = MP // BLK


def _dense_body(q_ref, bank_ref, sims_ref):
    qv = q_ref[0]
    qn = jnp.maximum(jnp.sqrt(jnp.sum(qv * qv)), 1e-8)
    for i in range(NBLK):
        n = BLK if i + 1 < NBLK else M - (NBLK - 1) * BLK
        x = bank_ref[pl.ds(i * BLK, n)]
        s = jnp.sum(x, axis=1)
        dot = jnp.sum(s * qv[None, :], axis=1)
        sq = jnp.sum(s * s, axis=1)
        norm = jnp.maximum(jnp.sqrt(sq) * (1.0 / A), 1e-8)
        sims = (dot * (1.0 / A)) / (norm * qn)
        if n < BLK:
            sims = jnp.pad(sims, (0, BLK - n),
                           constant_values=-jnp.inf)
        flat = i * BLK + lax.broadcasted_iota(jnp.int32, (BLK,), 0)
        sims = jnp.where(flat < M, sims, jnp.float32(-jnp.inf))
        sims_ref[i] = sims.reshape(1, BLK)


_dense_call = pl.pallas_call(
    _dense_body,
    in_specs=[
        pl.BlockSpec(memory_space=pltpu.MemorySpace.VMEM),
        pl.BlockSpec(memory_space=pltpu.MemorySpace.VMEM),
    ],
    out_specs=pl.BlockSpec(memory_space=pltpu.MemorySpace.VMEM),
    out_shape=jax.ShapeDtypeStruct((MP // BLK, 1, BLK), jnp.float32),
)


_MESH = plsc.VectorSubcoreMesh(core_axis_name="c", subcore_axis_name="s",
                               num_cores=1)


@functools.partial(
    pl.kernel,
    out_type=(
        jax.ShapeDtypeStruct((L,), jnp.float32),
        jax.ShapeDtypeStruct((L,), jnp.int32),
    ),
    mesh=_MESH,
    scratch_types=[
        pltpu.VMEM((MP,), jnp.float32),
        pltpu.VMEM((L,), jnp.float32),
        pltpu.VMEM((L,), jnp.int32),
    ],
    compiler_params=pltpu.CompilerParams(needs_layout_passes=False),
)
def _top_kernel(sims_hbm, tv_hbm, ti_hbm, sims_v, tv_v, ti_v):
    wid = lax.axis_index("s") * 2 + lax.axis_index("c")

    @pl.when(wid == 0)
    def _():
        pltpu.sync_copy(sims_hbm, sims_v)

        lanes = lax.iota(jnp.int32, L)
        neg = jnp.float32(-jnp.inf)
        big = jnp.int32(2**30)
        mx0 = jnp.full((L,), neg)
        mi0 = jnp.full((L,), big)

        def ins(v, gidx, st):
            v1, i1, v2, i2, v3, i3 = st
            c1 = v > v1
            nv1 = jnp.where(c1, v, v1)
            ni1 = jnp.where(c1, gidx, i1)
            dv = jnp.where(c1, v1, v)
            di = jnp.where(c1, i1, gidx)
            c2 = dv > v2
            nv2 = jnp.where(c2, dv, v2)
            ni2 = jnp.where(c2, di, i2)
            dv2 = jnp.where(c2, v2, dv)
            di2 = jnp.where(c2, i2, di)
            c3 = dv2 > v3
            nv3 = jnp.where(c3, dv2, v3)
            ni3 = jnp.where(c3, di2, i3)
            return (nv1, ni1, nv2, ni2, nv3, ni3)

        st0 = (mx0, mi0, mx0, mi0, mx0, mi0)

        def body(i, carry):
            a, b = carry
            ca = i * 2
            a = ins(sims_v[pl.ds(ca * L, L)], lanes + ca * L, a)
            b = ins(sims_v[pl.ds((ca + 1) * L, L)], lanes + (ca + 1) * L, b)
            return (a, b)

        a, b = lax.fori_loop(0, MP // L // 2, body, (st0, st0))
        cands = [(a[0], a[1]), (a[2], a[3]), (a[4], a[5]),
                 (b[0], b[1]), (b[2], b[3]), (b[4], b[5])]

        def finish(mx, mi):
            gmax = jnp.max(mx)
            gidx = jnp.min(jnp.where(mx == gmax, mi, big))
            return gmax, gidx

        def sel(prev):
            mx, mi = mx0, mi0
            for cv, ci in cands:
                if prev is not None:
                    gv, gi = prev
                    keep = (cv < gv) | ((cv == gv) & (ci > gi))
                    cv = jnp.where(keep, cv, neg)
                better = (cv > mx) | ((cv == mx) & (ci < mi))
                mx = jnp.where(better, cv, mx)
                mi = jnp.where(better, ci, mi)
            return finish(mx, mi)

        p0 = sel(None)
        p1 = sel(p0)
        p2 = sel(p1)

        tv = jnp.where(lanes == 0, p0[0],
                       jnp.where(lanes == 1, p1[0],
                                 jnp.where(lanes == 2, p2[0],
                                           jnp.float32(0.0))))
        ti = jnp.where(lanes == 0, p0[1],
                       jnp.where(lanes == 1, p1[1],
                                 jnp.where(lanes == 2, p2[1],
                                           jnp.int32(0))))
        tv_v[...] = tv
        ti_v[...] = ti
        pltpu.sync_copy(tv_v, tv_hbm)
        pltpu.sync_copy(ti_v, ti_hbm)


def _gather_body(ti_ref, bank_ref, best_ref):
    best_ref[...] = bank_ref[...]


_gather_call = pl.pallas_call(
    _gather_body,
    grid_spec=pltpu.PrefetchScalarGridSpec(
        num_scalar_prefetch=1,
        grid=(1,),
        in_specs=[
            pl.BlockSpec((1, A, D), lambda i, ti_ref: (ti_ref[0], 0, 0)),
        ],
        out_specs=pl.BlockSpec((1, A, D), lambda i, ti_ref: (0, 0, 0)),
    ),
    out_shape=jax.ShapeDtypeStruct((1, A, D), jnp.float32),
)


def kernel(query_embedding, memory_bank, k):
    sims8 = _dense_call(query_embedding.reshape(1, D), memory_bank)
    sims1 = sims8.reshape(MP)
    tv, ti = _top_kernel(sims1)
    best = _gather_call(ti, memory_bank)
    return (sims1[:M], tv[:K], ti[:K], best[0])
